# baseline (speedup 1.0000x reference)
import functools

import jax
import jax.numpy as jnp
from jax import lax
from jax.experimental import pallas as pl
from jax.experimental.pallas import tpu as pltpu
from jax.experimental.pallas import tpu_sc as plsc

N = 10000
E = 320000
D = 128
H = 64
C = 10
G = 128

NC = 2
NS = 16
NW = NC * NS
CH = 128
NCH = 80
MR = 3
MCH = MR * CH
K0 = 38
K1 = 16
KM = max(K0, K1)
EP = NW * NCH * CH
NP = 10240
RPT = NP // NS

BN = 1000

def _segment_sum_body(table, src_idx0, dst_idx0, src_idx1, dst_idx1, zeros,
                      out, src_v, dst_v, rows_v, acc,
                      gsem0, gsem1, ssem0, ssem1):
    cid = lax.axis_index("c")
    sid = lax.axis_index("s")

    @pl.when(cid == 0)
    def _():
        pltpu.sync_copy(src_idx0.at[sid], src_v.at[pl.ds(0, K0)])
        pltpu.sync_copy(dst_idx0.at[sid], dst_v.at[pl.ds(0, K0)])

    @pl.when(cid == 1)
    def _():
        pltpu.sync_copy(src_idx1.at[sid], src_v.at[pl.ds(0, K1)])
        pltpu.sync_copy(dst_idx1.at[sid], dst_v.at[pl.ds(0, K1)])

    pltpu.sync_copy(zeros, acc.at[pl.ds(sid * RPT, RPT)])
    plsc.subcore_barrier()

    def gather(j, slot, sem):
        pltpu.make_async_copy(table.at[src_v.at[j]],
                              rows_v.at[slot], sem).start()

    def gather_wait(j, slot, sem):
        pltpu.make_async_copy(table.at[src_v.at[j]],
                              rows_v.at[slot], sem).wait()

    def scatter(j, slot, sem):
        pltpu.async_copy(rows_v.at[slot],
                         acc.at[dst_v.at[j]], sem, add=True)

    def scatter_wait(j, slot, sem):
        pltpu.make_async_copy(rows_v.at[slot],
                              acc.at[dst_v.at[j]], sem).wait()

    gather(0, 0, gsem0)
    gather(1, 1, gsem1)

    n_mc = jnp.where(cid == 0, K0, K1)

    def body(jj, carry):
        j0 = jj * 2
        gather_wait(j0, 0, gsem0)
        scatter(j0, 0, ssem0)
        gather_wait(j0 + 1, 1, gsem1)
        scatter(j0 + 1, 1, ssem1)
        scatter_wait(j0, 0, ssem0)

        @pl.when(j0 + 2 < n_mc)
        def _():
            gather(j0 + 2, 0, gsem0)

        scatter_wait(j0 + 1, 1, ssem1)

        @pl.when(j0 + 3 < n_mc)
        def _():
            gather(j0 + 3, 1, gsem1)

        return carry

    lax.fori_loop(0, n_mc // 2, body, 0)
    plsc.subcore_barrier()
    pltpu.sync_copy(acc.at[pl.ds(sid * RPT, RPT)],
                    out.at[cid, pl.ds(sid * RPT, RPT)])


@functools.cache
def _segment_sum_sc():
    mesh = plsc.VectorSubcoreMesh(core_axis_name="c", subcore_axis_name="s")
    return pl.kernel(
        _segment_sum_body,
        mesh=mesh,
        out_type=jax.ShapeDtypeStruct((NC, NP, H), jnp.float32),
        scratch_types=[
            pltpu.VMEM((KM, MCH), jnp.int32),
            pltpu.VMEM((KM, MCH), jnp.int32),
            pltpu.VMEM((2, MCH, H), jnp.float32),
            pltpu.VMEM_SHARED((NP, H), jnp.float32),
            pltpu.SemaphoreType.DMA,
            pltpu.SemaphoreType.DMA,
            pltpu.SemaphoreType.DMA,
            pltpu.SemaphoreType.DMA,
        ],
        compiler_params=pltpu.CompilerParams(use_tc_tiling_on_sc=False),
    )


def _proj2_body(x_ref, wa_ref, wb_ref, oa_ref, ob_ref):
    x = x_ref[...]
    dn = (((1,), (1,)), ((), ()))
    oa_ref[...] = lax.dot_general(x, wa_ref[...], dn,
                                  preferred_element_type=jnp.float32)
    ob_ref[...] = lax.dot_general(x, wb_ref[...], dn,
                                  preferred_element_type=jnp.float32)


def _proj2(x, wa, wb):
    d = x.shape[1]
    return pl.pallas_call(
        _proj2_body,
        grid=(N // BN,),
        in_specs=[
            pl.BlockSpec((BN, d), lambda i: (i, 0)),
            pl.BlockSpec((H, d), lambda i: (0, 0)),
            pl.BlockSpec((H, d), lambda i: (0, 0)),
        ],
        out_specs=[
            pl.BlockSpec((BN, H), lambda i: (i, 0)),
            pl.BlockSpec((BN, H), lambda i: (i, 0)),
        ],
        out_shape=[
            jax.ShapeDtypeStruct((N, H), jnp.float32),
            jax.ShapeDtypeStruct((N, H), jnp.float32),
        ],
    )(x, wa, wb)


def _layer_body(agg_ref, xr_ref, b_ref, wa_ref, wb_ref, oa_ref, ob_ref):
    h = jnp.maximum(agg_ref[0] + agg_ref[1] + xr_ref[...] + b_ref[...], 0.0)
    dn = (((1,), (1,)), ((), ()))
    oa_ref[...] = lax.dot_general(h, wa_ref[...], dn,
                                  preferred_element_type=jnp.float32)
    ob_ref[...] = lax.dot_general(h, wb_ref[...], dn,
                                  preferred_element_type=jnp.float32)


def _layer(agg, xr, b, wa, wb):
    return pl.pallas_call(
        _layer_body,
        grid=(N // BN,),
        in_specs=[
            pl.BlockSpec((NC, BN, H), lambda i: (0, i, 0)),
            pl.BlockSpec((BN, H), lambda i: (i, 0)),
            pl.BlockSpec((1, H), lambda i: (0, 0)),
            pl.BlockSpec((H, H), lambda i: (0, 0)),
            pl.BlockSpec((H, H), lambda i: (0, 0)),
        ],
        out_specs=[
            pl.BlockSpec((BN, H), lambda i: (i, 0)),
            pl.BlockSpec((BN, H), lambda i: (i, 0)),
        ],
        out_shape=[
            jax.ShapeDtypeStruct((N, H), jnp.float32),
            jax.ShapeDtypeStruct((N, H), jnp.float32),
        ],
    )(agg, xr, b, wa, wb)


def _pool_body(agg_ref, xr_ref, b_ref, batch_ref, wc_ref, bc_ref, out_ref,
               acc_ref):
    i = pl.program_id(0)

    @pl.when(i == 0)
    def _():
        acc_ref[...] = jnp.zeros_like(acc_ref)

    h = jnp.maximum(agg_ref[0] + agg_ref[1] + xr_ref[...] + b_ref[...], 0.0)
    ext = jnp.concatenate([h, jnp.ones((BN, 1), jnp.float32)], axis=1)
    onehot = (batch_ref[...] ==
              lax.broadcasted_iota(jnp.int32, (BN, G), 1)).astype(jnp.float32)
    acc_ref[...] += lax.dot_general(onehot, ext, (((0,), (0,)), ((), ())),
                                    preferred_element_type=jnp.float32)

    @pl.when(i == pl.num_programs(0) - 1)
    def _():
        sums = acc_ref[:, :H]
        cnt = acc_ref[:, H:H + 1]
        g = sums / jnp.maximum(cnt, 1.0)
        out_ref[...] = lax.dot_general(g, wc_ref[...], (((1,), (1,)), ((), ())),
                                       preferred_element_type=jnp.float32) \
            + bc_ref[...]


def _pool(agg, xr, b, batch2d, wc, bc2d):
    return pl.pallas_call(
        _pool_body,
        grid=(N // BN,),
        in_specs=[
            pl.BlockSpec((NC, BN, H), lambda i: (0, i, 0)),
            pl.BlockSpec((BN, H), lambda i: (i, 0)),
            pl.BlockSpec((1, H), lambda i: (0, 0)),
            pl.BlockSpec((BN, 1), lambda i: (i, 0)),
            pl.BlockSpec((C, H), lambda i: (0, 0)),
            pl.BlockSpec((1, C), lambda i: (0, 0)),
        ],
        out_specs=pl.BlockSpec((G, C), lambda i: (0, 0)),
        out_shape=jax.ShapeDtypeStruct((G, C), jnp.float32),
        scratch_shapes=[pltpu.VMEM((G, H + 1), jnp.float32)],
    )(agg, xr, b, batch2d, wc, bc2d)


def kernel(x, edge_index, batch, W1_rel, b1, W1_root, W2_rel, b2, W2_root,
           Wc, bc):
    src = edge_index[0]
    dst = edge_index[1]
    cap0, cap1 = NS * K0 * MCH, NS * K1 * MCH
    pad = cap0 + cap1 - E
    def slabs(idx, padval):
        full = jnp.concatenate([idx, jnp.full((pad,), padval, jnp.int32)])
        return (full[:cap0].reshape(NS, K0, MCH),
                full[cap0:].reshape(NS, K1, MCH))
    srcp0, srcp1 = slabs(src, 0)
    dstp0, dstp1 = slabs(dst, N)
    zeros = jnp.zeros((RPT, H), jnp.float32)

    seg = _segment_sum_sc()
    xw1, xr1 = _proj2(x, W1_rel, W1_root)
    agg1 = seg(xw1, srcp0, dstp0, srcp1, dstp1, zeros)
    h1w2, h1r2 = _layer(agg1, xr1, b1.reshape(1, H), W2_rel, W2_root)
    agg2 = seg(h1w2, srcp0, dstp0, srcp1, dstp1, zeros)
    return _pool(agg2, h1r2, b2.reshape(1, H), batch.reshape(N, 1), Wc,
                 bc.reshape(1, C))


# speedup vs baseline: 1.3591x; 1.3591x over previous
import functools

import jax
import jax.numpy as jnp
from jax import lax
from jax.experimental import pallas as pl
from jax.experimental.pallas import tpu as pltpu
from jax.experimental.pallas import tpu_sc as plsc

N = 10000
E = 320000
D = 128
H = 64
C = 10
G = 128

NC = 2
NS = 16
NW = NC * NS
CH = 128
NCH = 80
MR = 2
MCH = MR * CH
K0 = 40
K1 = 40
KM = max(K0, K1)
NSLOT = 4
EP = NW * NCH * CH
NP = 10240
RPT = NP // NS

BN = 1000

def _segment_sum_body(table, src_idx0, dst_idx0, src_idx1, dst_idx1, zeros,
                      out, src_v, dst_v, rows_v, acc, *sems):
    gsems = sems[:NSLOT]
    ssems = sems[NSLOT:]
    cid = lax.axis_index("c")
    sid = lax.axis_index("s")

    @pl.when(cid == 0)
    def _():
        pltpu.sync_copy(src_idx0.at[sid], src_v.at[pl.ds(0, K0)])
        pltpu.sync_copy(dst_idx0.at[sid], dst_v.at[pl.ds(0, K0)])

    @pl.when(cid == 1)
    def _():
        pltpu.sync_copy(src_idx1.at[sid], src_v.at[pl.ds(0, K1)])
        pltpu.sync_copy(dst_idx1.at[sid], dst_v.at[pl.ds(0, K1)])

    pltpu.sync_copy(zeros, acc.at[pl.ds(sid * RPT, RPT)])
    plsc.subcore_barrier()

    def gather(j, slot, sem):
        pltpu.make_async_copy(table.at[src_v.at[j]],
                              rows_v.at[slot], sem).start()

    def gather_wait(j, slot, sem):
        pltpu.make_async_copy(table.at[src_v.at[j]],
                              rows_v.at[slot], sem).wait()

    def scatter(j, slot, sem):
        pltpu.async_copy(rows_v.at[slot],
                         acc.at[dst_v.at[j]], sem, add=True)

    def scatter_wait(j, slot, sem):
        pltpu.make_async_copy(rows_v.at[slot],
                              acc.at[dst_v.at[j]], sem).wait()

    n_mc = jnp.where(cid == 0, K0, K1)

    for sl in range(NSLOT):
        gather(sl, sl, gsems[sl])

    def body(jj, carry):
        j0 = jj * NSLOT
        for sl in range(NSLOT):
            gather_wait(j0 + sl, sl, gsems[sl])
            scatter(j0 + sl, sl, ssems[sl])
        for sl in range(NSLOT):
            scatter_wait(j0 + sl, sl, ssems[sl])

            @pl.when(j0 + NSLOT + sl < n_mc)
            def _():
                gather(j0 + NSLOT + sl, sl, gsems[sl])

        return carry

    lax.fori_loop(0, n_mc // NSLOT, body, 0)
    plsc.subcore_barrier()
    pltpu.sync_copy(acc.at[pl.ds(sid * RPT, RPT)],
                    out.at[cid, pl.ds(sid * RPT, RPT)])


@functools.cache
def _segment_sum_sc():
    mesh = plsc.VectorSubcoreMesh(core_axis_name="c", subcore_axis_name="s")
    return pl.kernel(
        _segment_sum_body,
        mesh=mesh,
        out_type=jax.ShapeDtypeStruct((NC, NP, H), jnp.float32),
        scratch_types=[
            pltpu.VMEM((KM, MCH), jnp.int32),
            pltpu.VMEM((KM, MCH), jnp.int32),
            pltpu.VMEM((NSLOT, MCH, H), jnp.float32),
            pltpu.VMEM_SHARED((NP, H), jnp.float32),
        ] + [pltpu.SemaphoreType.DMA] * (2 * NSLOT),
        compiler_params=pltpu.CompilerParams(use_tc_tiling_on_sc=False),
    )


def _proj2_body(x_ref, wa_ref, wb_ref, oa_ref, ob_ref):
    x = x_ref[...]
    dn = (((1,), (1,)), ((), ()))
    oa_ref[...] = lax.dot_general(x, wa_ref[...], dn,
                                  preferred_element_type=jnp.float32)
    ob_ref[...] = lax.dot_general(x, wb_ref[...], dn,
                                  preferred_element_type=jnp.float32)


def _proj2(x, wa, wb):
    d = x.shape[1]
    return pl.pallas_call(
        _proj2_body,
        grid=(N // BN,),
        in_specs=[
            pl.BlockSpec((BN, d), lambda i: (i, 0)),
            pl.BlockSpec((H, d), lambda i: (0, 0)),
            pl.BlockSpec((H, d), lambda i: (0, 0)),
        ],
        out_specs=[
            pl.BlockSpec((BN, H), lambda i: (i, 0)),
            pl.BlockSpec((BN, H), lambda i: (i, 0)),
        ],
        out_shape=[
            jax.ShapeDtypeStruct((N, H), jnp.float32),
            jax.ShapeDtypeStruct((N, H), jnp.float32),
        ],
    )(x, wa, wb)


def _layer_body(agg_ref, xr_ref, b_ref, wa_ref, wb_ref, oa_ref, ob_ref):
    h = jnp.maximum(agg_ref[0] + agg_ref[1] + xr_ref[...] + b_ref[...], 0.0)
    dn = (((1,), (1,)), ((), ()))
    oa_ref[...] = lax.dot_general(h, wa_ref[...], dn,
                                  preferred_element_type=jnp.float32)
    ob_ref[...] = lax.dot_general(h, wb_ref[...], dn,
                                  preferred_element_type=jnp.float32)


def _layer(agg, xr, b, wa, wb):
    return pl.pallas_call(
        _layer_body,
        grid=(N // BN,),
        in_specs=[
            pl.BlockSpec((NC, BN, H), lambda i: (0, i, 0)),
            pl.BlockSpec((BN, H), lambda i: (i, 0)),
            pl.BlockSpec((1, H), lambda i: (0, 0)),
            pl.BlockSpec((H, H), lambda i: (0, 0)),
            pl.BlockSpec((H, H), lambda i: (0, 0)),
        ],
        out_specs=[
            pl.BlockSpec((BN, H), lambda i: (i, 0)),
            pl.BlockSpec((BN, H), lambda i: (i, 0)),
        ],
        out_shape=[
            jax.ShapeDtypeStruct((N, H), jnp.float32),
            jax.ShapeDtypeStruct((N, H), jnp.float32),
        ],
    )(agg, xr, b, wa, wb)


def _pool_body(agg_ref, xr_ref, b_ref, batch_ref, wc_ref, bc_ref, out_ref,
               acc_ref):
    i = pl.program_id(0)

    @pl.when(i == 0)
    def _():
        acc_ref[...] = jnp.zeros_like(acc_ref)

    h = jnp.maximum(agg_ref[0] + agg_ref[1] + xr_ref[...] + b_ref[...], 0.0)
    ext = jnp.concatenate([h, jnp.ones((BN, 1), jnp.float32)], axis=1)
    onehot = (batch_ref[...] ==
              lax.broadcasted_iota(jnp.int32, (BN, G), 1)).astype(jnp.float32)
    acc_ref[...] += lax.dot_general(onehot, ext, (((0,), (0,)), ((), ())),
                                    preferred_element_type=jnp.float32)

    @pl.when(i == pl.num_programs(0) - 1)
    def _():
        sums = acc_ref[:, :H]
        cnt = acc_ref[:, H:H + 1]
        g = sums / jnp.maximum(cnt, 1.0)
        out_ref[...] = lax.dot_general(g, wc_ref[...], (((1,), (1,)), ((), ())),
                                       preferred_element_type=jnp.float32) \
            + bc_ref[...]


def _pool(agg, xr, b, batch2d, wc, bc2d):
    return pl.pallas_call(
        _pool_body,
        grid=(N // BN,),
        in_specs=[
            pl.BlockSpec((NC, BN, H), lambda i: (0, i, 0)),
            pl.BlockSpec((BN, H), lambda i: (i, 0)),
            pl.BlockSpec((1, H), lambda i: (0, 0)),
            pl.BlockSpec((BN, 1), lambda i: (i, 0)),
            pl.BlockSpec((C, H), lambda i: (0, 0)),
            pl.BlockSpec((1, C), lambda i: (0, 0)),
        ],
        out_specs=pl.BlockSpec((G, C), lambda i: (0, 0)),
        out_shape=jax.ShapeDtypeStruct((G, C), jnp.float32),
        scratch_shapes=[pltpu.VMEM((G, H + 1), jnp.float32)],
    )(agg, xr, b, batch2d, wc, bc2d)


def kernel(x, edge_index, batch, W1_rel, b1, W1_root, W2_rel, b2, W2_root,
           Wc, bc):
    src = edge_index[0]
    dst = edge_index[1]
    cap0, cap1 = NS * K0 * MCH, NS * K1 * MCH
    pad = cap0 + cap1 - E
    def slabs(idx, padval):
        full = jnp.concatenate([idx, jnp.full((pad,), padval, jnp.int32)])
        return (full[:cap0].reshape(NS, K0, MCH),
                full[cap0:].reshape(NS, K1, MCH))
    srcp0, srcp1 = slabs(src, 0)
    dstp0, dstp1 = slabs(dst, N)
    zeros = jnp.zeros((RPT, H), jnp.float32)

    seg = _segment_sum_sc()
    xw1, xr1 = _proj2(x, W1_rel, W1_root)
    agg1 = seg(xw1, srcp0, dstp0, srcp1, dstp1, zeros)
    h1w2, h1r2 = _layer(agg1, xr1, b1.reshape(1, H), W2_rel, W2_root)
    agg2 = seg(h1w2, srcp0, dstp0, srcp1, dstp1, zeros)
    return _pool(agg2, h1r2, b2.reshape(1, H), batch.reshape(N, 1), Wc,
                 bc.reshape(1, C))


# speedup vs baseline: 1.5288x; 1.1248x over previous
import functools

import jax
import jax.numpy as jnp
from jax import lax
from jax.experimental import pallas as pl
from jax.experimental.pallas import tpu as pltpu
from jax.experimental.pallas import tpu_sc as plsc

N = 10000
E = 320000
D = 128
H = 64
C = 10
G = 128

NC = 2
NS = 16
NW = NC * NS
CH = 128
NCH = 80
MR = 2
MCH = MR * CH
K0 = 78
K1 = 2
KM = max(K0, K1)
NSLOT = 2
EP = NW * NCH * CH
NP = 10240
RPT = NP // NS

BN = 1000

def _segment_sum_body(table, src_idx0, dst_idx0, src_idx1, dst_idx1, zeros,
                      out, src_v, dst_v, rows_v, acc, *sems):
    gsems = sems[:NSLOT]
    ssems = sems[NSLOT:]
    cid = lax.axis_index("c")
    sid = lax.axis_index("s")

    @pl.when(cid == 0)
    def _():
        pltpu.sync_copy(src_idx0.at[sid], src_v.at[pl.ds(0, K0)])
        pltpu.sync_copy(dst_idx0.at[sid], dst_v.at[pl.ds(0, K0)])

    @pl.when(cid == 1)
    def _():
        pltpu.sync_copy(src_idx1.at[sid], src_v.at[pl.ds(0, K1)])
        pltpu.sync_copy(dst_idx1.at[sid], dst_v.at[pl.ds(0, K1)])

    pltpu.sync_copy(zeros, acc.at[pl.ds(sid * RPT, RPT)])
    plsc.subcore_barrier()

    def gather(j, slot, sem):
        pltpu.make_async_copy(table.at[src_v.at[j]],
                              rows_v.at[slot], sem).start()

    def gather_wait(j, slot, sem):
        pltpu.make_async_copy(table.at[src_v.at[j]],
                              rows_v.at[slot], sem).wait()

    def scatter(j, slot, sem):
        pltpu.async_copy(rows_v.at[slot],
                         acc.at[dst_v.at[j]], sem, add=True)

    def scatter_wait(j, slot, sem):
        pltpu.make_async_copy(rows_v.at[slot],
                              acc.at[dst_v.at[j]], sem).wait()

    n_mc = jnp.where(cid == 0, K0, K1)

    for sl in range(NSLOT):
        @pl.when(sl < n_mc)
        def _():
            gather(sl, sl, gsems[sl])

    def body(jj, carry):
        j0 = jj * NSLOT
        for sl in range(NSLOT):
            gather_wait(j0 + sl, sl, gsems[sl])
            scatter(j0 + sl, sl, ssems[sl])
        for sl in range(NSLOT):
            scatter_wait(j0 + sl, sl, ssems[sl])

            @pl.when(j0 + NSLOT + sl < n_mc)
            def _():
                gather(j0 + NSLOT + sl, sl, gsems[sl])

        return carry

    lax.fori_loop(0, n_mc // NSLOT, body, 0)
    plsc.subcore_barrier()
    pltpu.sync_copy(acc.at[pl.ds(sid * RPT, RPT)],
                    out.at[cid, pl.ds(sid * RPT, RPT)])


@functools.cache
def _segment_sum_sc():
    mesh = plsc.VectorSubcoreMesh(core_axis_name="c", subcore_axis_name="s")
    return pl.kernel(
        _segment_sum_body,
        mesh=mesh,
        out_type=jax.ShapeDtypeStruct((NC, NP, H), jnp.float32),
        scratch_types=[
            pltpu.VMEM((KM, MCH), jnp.int32),
            pltpu.VMEM((KM, MCH), jnp.int32),
            pltpu.VMEM((NSLOT, MCH, H), jnp.float32),
            pltpu.VMEM_SHARED((NP, H), jnp.float32),
        ] + [pltpu.SemaphoreType.DMA] * (2 * NSLOT),
        compiler_params=pltpu.CompilerParams(use_tc_tiling_on_sc=False),
    )


def _proj2_body(x_ref, wa_ref, wb_ref, oa_ref, ob_ref):
    x = x_ref[...]
    dn = (((1,), (1,)), ((), ()))
    oa_ref[...] = lax.dot_general(x, wa_ref[...], dn,
                                  preferred_element_type=jnp.float32)
    ob_ref[...] = lax.dot_general(x, wb_ref[...], dn,
                                  preferred_element_type=jnp.float32)


def _proj2(x, wa, wb):
    d = x.shape[1]
    return pl.pallas_call(
        _proj2_body,
        grid=(N // BN,),
        in_specs=[
            pl.BlockSpec((BN, d), lambda i: (i, 0)),
            pl.BlockSpec((H, d), lambda i: (0, 0)),
            pl.BlockSpec((H, d), lambda i: (0, 0)),
        ],
        out_specs=[
            pl.BlockSpec((BN, H), lambda i: (i, 0)),
            pl.BlockSpec((BN, H), lambda i: (i, 0)),
        ],
        out_shape=[
            jax.ShapeDtypeStruct((N, H), jnp.float32),
            jax.ShapeDtypeStruct((N, H), jnp.float32),
        ],
    )(x, wa, wb)


def _layer_body(agg_ref, xr_ref, b_ref, wa_ref, wb_ref, oa_ref, ob_ref):
    h = jnp.maximum(agg_ref[0] + agg_ref[1] + xr_ref[...] + b_ref[...], 0.0)
    dn = (((1,), (1,)), ((), ()))
    oa_ref[...] = lax.dot_general(h, wa_ref[...], dn,
                                  preferred_element_type=jnp.float32)
    ob_ref[...] = lax.dot_general(h, wb_ref[...], dn,
                                  preferred_element_type=jnp.float32)


def _layer(agg, xr, b, wa, wb):
    return pl.pallas_call(
        _layer_body,
        grid=(N // BN,),
        in_specs=[
            pl.BlockSpec((NC, BN, H), lambda i: (0, i, 0)),
            pl.BlockSpec((BN, H), lambda i: (i, 0)),
            pl.BlockSpec((1, H), lambda i: (0, 0)),
            pl.BlockSpec((H, H), lambda i: (0, 0)),
            pl.BlockSpec((H, H), lambda i: (0, 0)),
        ],
        out_specs=[
            pl.BlockSpec((BN, H), lambda i: (i, 0)),
            pl.BlockSpec((BN, H), lambda i: (i, 0)),
        ],
        out_shape=[
            jax.ShapeDtypeStruct((N, H), jnp.float32),
            jax.ShapeDtypeStruct((N, H), jnp.float32),
        ],
    )(agg, xr, b, wa, wb)


def _pool_body(agg_ref, xr_ref, b_ref, batch_ref, wc_ref, bc_ref, out_ref,
               acc_ref):
    i = pl.program_id(0)

    @pl.when(i == 0)
    def _():
        acc_ref[...] = jnp.zeros_like(acc_ref)

    h = jnp.maximum(agg_ref[0] + agg_ref[1] + xr_ref[...] + b_ref[...], 0.0)
    ext = jnp.concatenate([h, jnp.ones((BN, 1), jnp.float32)], axis=1)
    onehot = (batch_ref[...] ==
              lax.broadcasted_iota(jnp.int32, (BN, G), 1)).astype(jnp.float32)
    acc_ref[...] += lax.dot_general(onehot, ext, (((0,), (0,)), ((), ())),
                                    preferred_element_type=jnp.float32)

    @pl.when(i == pl.num_programs(0) - 1)
    def _():
        sums = acc_ref[:, :H]
        cnt = acc_ref[:, H:H + 1]
        g = sums / jnp.maximum(cnt, 1.0)
        out_ref[...] = lax.dot_general(g, wc_ref[...], (((1,), (1,)), ((), ())),
                                       preferred_element_type=jnp.float32) \
            + bc_ref[...]


def _pool(agg, xr, b, batch2d, wc, bc2d):
    return pl.pallas_call(
        _pool_body,
        grid=(N // BN,),
        in_specs=[
            pl.BlockSpec((NC, BN, H), lambda i: (0, i, 0)),
            pl.BlockSpec((BN, H), lambda i: (i, 0)),
            pl.BlockSpec((1, H), lambda i: (0, 0)),
            pl.BlockSpec((BN, 1), lambda i: (i, 0)),
            pl.BlockSpec((C, H), lambda i: (0, 0)),
            pl.BlockSpec((1, C), lambda i: (0, 0)),
        ],
        out_specs=pl.BlockSpec((G, C), lambda i: (0, 0)),
        out_shape=jax.ShapeDtypeStruct((G, C), jnp.float32),
        scratch_shapes=[pltpu.VMEM((G, H + 1), jnp.float32)],
    )(agg, xr, b, batch2d, wc, bc2d)


def kernel(x, edge_index, batch, W1_rel, b1, W1_root, W2_rel, b2, W2_root,
           Wc, bc):
    src = edge_index[0]
    dst = edge_index[1]
    cap0, cap1 = NS * K0 * MCH, NS * K1 * MCH
    pad = cap0 + cap1 - E
    def slabs(idx, padval):
        full = jnp.concatenate([idx, jnp.full((pad,), padval, jnp.int32)])
        return (full[:cap0].reshape(NS, K0, MCH),
                full[cap0:].reshape(NS, K1, MCH))
    srcp0, srcp1 = slabs(src, 0)
    dstp0, dstp1 = slabs(dst, N)
    zeros = jnp.zeros((RPT, H), jnp.float32)

    seg = _segment_sum_sc()
    xw1, xr1 = _proj2(x, W1_rel, W1_root)
    agg1 = seg(xw1, srcp0, dstp0, srcp1, dstp1, zeros)
    h1w2, h1r2 = _layer(agg1, xr1, b1.reshape(1, H), W2_rel, W2_root)
    agg2 = seg(h1w2, srcp0, dstp0, srcp1, dstp1, zeros)
    return _pool(agg2, h1r2, b2.reshape(1, H), batch.reshape(N, 1), Wc,
                 bc.reshape(1, C))
